# baseline (speedup 1.0000x reference)
import functools
import jax
import jax.numpy as jnp
from jax import lax
from jax.experimental import pallas as pl
from jax.experimental.pallas import tpu as pltpu
from jax.experimental.pallas import tpu_sc as plsc

N = 10000
D = 128
R = 19
E = 320000
NC = 2
NS = 16
LN = 16
NW = NC * NS
N_PAD = 10240
E_PAD = NW * 10240
EPT = E_PAD // NW
SUB = 128
W1 = 2048
W2 = 128
NBLK = 10
BN = N // NBLK


def _tca1_body(x_ref, w_ref, q_ref, k_ref, qh_ref, kh_ref):
    xb = x_ref[...]
    wf = w_ref[...].reshape(R * D, D)
    wq = jnp.dot(wf, q_ref[...], preferred_element_type=jnp.float32)
    wk = jnp.dot(wf, k_ref[...], preferred_element_type=jnp.float32)
    qh_ref[...] = jnp.dot(xb, wq.reshape(R, D).T,
                          preferred_element_type=jnp.float32)
    kh_ref[...] = jnp.dot(xb, wk.reshape(R, D).T,
                          preferred_element_type=jnp.float32)


def _tc_a1(x, weight, q, k):
    return pl.pallas_call(
        _tca1_body,
        grid=(NBLK,),
        in_specs=[
            pl.BlockSpec((BN, D), lambda i: (i, 0)),
            pl.BlockSpec((R, D, D), lambda i: (0, 0, 0)),
            pl.BlockSpec((D, 1), lambda i: (0, 0)),
            pl.BlockSpec((D, 1), lambda i: (0, 0)),
        ],
        out_specs=[
            pl.BlockSpec((BN, R), lambda i: (i, 0)),
            pl.BlockSpec((BN, R), lambda i: (i, 0)),
        ],
        out_shape=[
            jax.ShapeDtypeStruct((N, R), jnp.float32),
            jax.ShapeDtypeStruct((N, R), jnp.float32),
        ],
    )(x, weight, q, k)


def _tca2_body(x_ref, w_ref, h_ref):
    xb = x_ref[...]
    for r in range(R):
        h_ref[r] = jnp.dot(xb, w_ref[r], preferred_element_type=jnp.float32)


def _tc_a2(x, weight):
    return pl.pallas_call(
        _tca2_body,
        grid=(NBLK,),
        in_specs=[
            pl.BlockSpec((BN, D), lambda i: (i, 0)),
            pl.BlockSpec((R, D, D), lambda i: (0, 0, 0)),
        ],
        out_specs=pl.BlockSpec((R, BN, D), lambda i: (0, i, 0)),
        out_shape=jax.ShapeDtypeStruct((R, N, D), jnp.float32),
    )(x, weight)


def _tcc_body(a0_ref, a1_ref, x_ref, b_ref, o_ref):
    o_ref[...] = a0_ref[...] + a1_ref[...] + x_ref[...] + b_ref[...]


def _tc_c(a0, a1, x, brow):
    return pl.pallas_call(
        _tcc_body,
        grid=(NBLK,),
        in_specs=[
            pl.BlockSpec((BN, D), lambda i: (i, 0)),
            pl.BlockSpec((BN, D), lambda i: (i, 0)),
            pl.BlockSpec((BN, D), lambda i: (i, 0)),
            pl.BlockSpec((1, D), lambda i: (0, 0)),
        ],
        out_specs=pl.BlockSpec((BN, D), lambda i: (i, 0)),
        out_shape=jax.ShapeDtypeStruct((N, D), jnp.float32),
    )(a0, a1, x, brow)


def _sc1_body(qh_hbm, kh_hbm, src_hbm, dst_hbm, et_hbm,
              ex_hbm, dp_hbm, ir_hbm,
              src_v, dst_v, et_v, idxq_v, idxk_v, idxr_v, qv_v, kv_v, ex_v,
              dst2_v, ex2_v, zb_v, den_sh, sem, sem2):
    cid = lax.axis_index("c")
    sid = lax.axis_index("s")
    wid = cid * NS + sid
    base = wid * EPT
    zseg = N_PAD // NS

    for i in range(zseg // LN):
        zb_v[pl.ds(i * LN, LN)] = jnp.zeros((LN,), jnp.float32)
    pltpu.sync_copy(zb_v, den_sh.at[pl.ds(sid * zseg, zseg)])
    plsc.subcore_barrier()

    def window(g, carry):
        b0 = base + g * W1
        lcps = [pltpu.async_copy(src_hbm.at[pl.ds(b0, W1)], src_v, sem),
                pltpu.async_copy(dst_hbm.at[pl.ds(b0, W1)], dst_v, sem),
                pltpu.async_copy(et_hbm.at[pl.ds(b0, W1)], et_v, sem)]
        for c in lcps:
            c.wait()

        def idx_body(c, cr):
            s16 = src_v[pl.ds(c * LN, LN)]
            d16 = dst_v[pl.ds(c * LN, LN)]
            t16 = et_v[pl.ds(c * LN, LN)]
            dcl = jnp.where(d16 < N, d16, d16 - N)
            idxq_v[pl.ds(c * LN, LN)] = dcl * R + t16
            idxk_v[pl.ds(c * LN, LN)] = s16 * R + t16
            idxr_v[pl.ds(c * LN, LN)] = t16 * N + s16
            return cr
        lax.fori_loop(0, W1 // LN, idx_body, 0, unroll=4)

        cps = []
        for j in range(W1 // SUB):
            cps.append(pltpu.async_copy(
                qh_hbm.at[idxq_v.at[pl.ds(j * SUB, SUB)]],
                qv_v.at[pl.ds(j * SUB, SUB)], sem))
            cps.append(pltpu.async_copy(
                kh_hbm.at[idxk_v.at[pl.ds(j * SUB, SUB)]],
                kv_v.at[pl.ds(j * SUB, SUB)], sem))
        cps.append(pltpu.async_copy(idxr_v, ir_hbm.at[pl.ds(b0, W1)], sem))
        for c in cps:
            c.wait()

        def ex_body(c, cr):
            a = qv_v[pl.ds(c * LN, LN)] + kv_v[pl.ds(c * LN, LN)]
            a = jnp.where(a >= 0.0, a, a * jnp.float32(0.2))
            ex_v[pl.ds(c * LN, LN)] = jnp.exp(a)
            return cr
        lax.fori_loop(0, W1 // LN, ex_body, 0, unroll=4)

        pltpu.sync_copy(ex_v, ex_hbm.at[pl.ds(b0, W1)])

        @pl.when(g > 0)
        def _():
            for j in range(W1 // SUB):
                pltpu.make_async_copy(
                    ex2_v.at[j], den_sh.at[dst2_v.at[j]], sem2).wait()

        for j in range(W1 // SUB):
            for c in range(SUB // LN):
                off = j * SUB + c * LN
                dst2_v[j, pl.ds(c * LN, LN)] = dst_v[pl.ds(off, LN)]
                ex2_v[j, pl.ds(c * LN, LN)] = ex_v[pl.ds(off, LN)]
        for j in range(W1 // SUB):
            pltpu.async_copy(ex2_v.at[j], den_sh.at[dst2_v.at[j]], sem2,
                             add=True)
        return carry

    lax.fori_loop(0, EPT // W1, window, 0)
    for j in range(W1 // SUB):
        pltpu.make_async_copy(ex2_v.at[j], den_sh.at[dst2_v.at[j]],
                              sem2).wait()
    plsc.subcore_barrier()
    pltpu.sync_copy(den_sh.at[pl.ds(sid * zseg, zseg)],
                    dp_hbm.at[pl.ds(cid * N_PAD + sid * zseg, zseg)])


def _sc1(qhf, khf, src_p, dst_p, et_p):
    mesh = plsc.VectorSubcoreMesh(core_axis_name="c", subcore_axis_name="s")
    f = functools.partial(
        pl.kernel, _sc1_body, mesh=mesh,
        out_type=(jax.ShapeDtypeStruct((E_PAD,), jnp.float32),
                  jax.ShapeDtypeStruct((NC * N_PAD,), jnp.float32),
                  jax.ShapeDtypeStruct((E_PAD,), jnp.int32)),
        scratch_types=[
            pltpu.VMEM((W1,), jnp.int32),
            pltpu.VMEM((W1,), jnp.int32),
            pltpu.VMEM((W1,), jnp.int32),
            pltpu.VMEM((W1,), jnp.int32),
            pltpu.VMEM((W1,), jnp.int32),
            pltpu.VMEM((W1,), jnp.int32),
            pltpu.VMEM((W1,), jnp.float32),
            pltpu.VMEM((W1,), jnp.float32),
            pltpu.VMEM((W1,), jnp.float32),
            pltpu.VMEM((W1 // SUB, SUB), jnp.int32),
            pltpu.VMEM((W1 // SUB, SUB), jnp.float32),
            pltpu.VMEM((N_PAD // NS,), jnp.float32),
            pltpu.VMEM_SHARED((N_PAD,), jnp.float32),
            pltpu.SemaphoreType.DMA,
            pltpu.SemaphoreType.DMA,
        ])()
    return f(qhf, khf, src_p, dst_p, et_p)


def _sc2_body(h_hbm, ex_hbm, dst_hbm, ir_hbm, dp2_hbm,
              al_hbm, ap_hbm,
              dn_v, rows_a, rows_b, ir_va, ir_vb, dst_va, dst_vb,
              ex_va, ex_vb, al_va, al_vb, ds_va, ds_vb, agg_sh,
              semd, semg0, semg1, sems0, sems1, sema0, sema1, seml0, seml1):
    cid = lax.axis_index("c")
    sid = lax.axis_index("s")
    wid = cid * NS + sid
    base = wid * EPT
    zseg = N_PAD // NS
    nwin = EPT // W2
    npair = nwin // 2

    nr = N_PAD // D
    cdn = pltpu.async_copy(dp2_hbm.at[pl.ds(0, nr)],
                           rows_b.at[pl.ds(0, nr)], semd)
    pltpu.sync_copy(dp2_hbm.at[pl.ds(nr, nr)], rows_a.at[pl.ds(0, nr)])
    cdn.wait()

    def dn_body(r, cr):
        for cc in range(D // LN):
            o = pl.ds(cc * LN, LN)
            dn_v[pl.ds(r * D + cc * LN, LN)] = jnp.float32(1.0) / (
                rows_b[r, o] + rows_a[r, o] + jnp.float32(1e-16))
        return cr
    lax.fori_loop(0, nr, dn_body, 0)

    def z_body(i, cr):
        for c in range(D // LN):
            rows_a[i, pl.ds(c * LN, LN)] = jnp.zeros((LN,), jnp.float32)
        return cr
    lax.fori_loop(0, W2, z_body, 0)
    for off in range(0, zseg, W2):
        pltpu.sync_copy(rows_a, agg_sh.at[pl.ds(sid * zseg + off, W2)])
    plsc.subcore_barrier()

    rows = (rows_a, rows_b)
    ir_v = (ir_va, ir_vb)
    dst_v = (dst_va, dst_vb)
    ex_v = (ex_va, ex_vb)
    al_v = (al_va, al_vb)
    ds_v = (ds_va, ds_vb)
    semg = (semg0, semg1)
    sems = (sems0, sems1)
    sema = (sema0, sema1)
    seml = (seml0, seml1)

    def lin_fire(g, p):
        b0 = base + g * W2
        pltpu.async_copy(ir_hbm.at[pl.ds(b0, W2)], ir_v[p], seml[p])
        pltpu.async_copy(dst_hbm.at[pl.ds(b0, W2)], dst_v[p], seml[p])
        pltpu.async_copy(ex_hbm.at[pl.ds(b0, W2)], ex_v[p], seml[p])

    def lin_wait(p):
        pltpu.make_async_copy(ir_hbm.at[pl.ds(base, W2)], ir_v[p], seml[p]).wait()
        pltpu.make_async_copy(dst_hbm.at[pl.ds(base, W2)], dst_v[p], seml[p]).wait()
        pltpu.make_async_copy(ex_hbm.at[pl.ds(base, W2)], ex_v[p], seml[p]).wait()

    def gather_fire(p):
        pltpu.async_copy(h_hbm.at[ir_v[p]], rows[p], semg[p])

    def gather_wait(p):
        pltpu.make_async_copy(h_hbm.at[ir_v[p]], rows[p], semg[p]).wait()

    def scat_fire(p):
        pltpu.async_copy(rows[p], agg_sh.at[ds_v[p]], sems[p], add=True)

    def scat_wait(p):
        pltpu.make_async_copy(rows[p], agg_sh.at[ds_v[p]], sems[p]).wait()

    def al_fire(g, p):
        pltpu.async_copy(al_v[p], al_hbm.at[pl.ds(base + g * W2, W2)], sema[p])

    def al_wait(p):
        pltpu.make_async_copy(al_v[p], al_hbm.at[pl.ds(base, W2)],
                              sema[p]).wait()

    lin_fire(0, 0)
    lin_fire(1, 1)
    lin_wait(0)
    gather_fire(0)

    def pair(i, carry):
        for b in range(2):
            p = b
            g = 2 * i + b
            gather_wait(p)
            if b == 0:
                @pl.when(i > 0)
                def _():
                    scat_wait(1)
                lin_wait(1)
                gather_fire(1)
            else:
                @pl.when(i < npair - 1)
                def _():
                    scat_wait(0)
                    lin_wait(0)
                    gather_fire(0)

            @pl.when(i > 0)
            def _():
                al_wait(p)

            for c in range(W2 // LN):
                d16 = dst_v[p][pl.ds(c * LN, LN)]
                inv = plsc.load_gather(dn_v, [d16])
                al_v[p][pl.ds(c * LN, LN)] = ex_v[p][pl.ds(c * LN, LN)] * inv
                ds_v[p][pl.ds(c * LN, LN)] = d16
            al_fire(g, p)

            @pl.when(i < npair - 1)
            def _():
                lin_fire(g + 2, p)

            def s_body(r0, cr):
                s = plsc.load_gather(al_v[p], [jnp.full((LN,), 0, jnp.int32) + r0])
                for c in range(D // LN):
                    rows[p][r0, pl.ds(c * LN, LN)] = \
                        rows[p][r0, pl.ds(c * LN, LN)] * s
                return cr
            lax.fori_loop(0, W2, s_body, 0, unroll=4)
            scat_fire(p)
        return carry

    lax.fori_loop(0, npair, pair, 0)
    scat_wait(0)
    scat_wait(1)
    al_wait(0)
    al_wait(1)
    plsc.subcore_barrier()
    pltpu.sync_copy(agg_sh.at[pl.ds(sid * zseg, zseg)],
                    ap_hbm.at[pl.ds(cid * N_PAD + sid * zseg, zseg)])


def _sc2(h, ex, dst_p, ir, dp):
    mesh = plsc.VectorSubcoreMesh(core_axis_name="c", subcore_axis_name="s")
    f = functools.partial(
        pl.kernel, _sc2_body, mesh=mesh,
        compiler_params=pltpu.CompilerParams(needs_layout_passes=False),
        out_type=(jax.ShapeDtypeStruct((E_PAD,), jnp.float32),
                  jax.ShapeDtypeStruct((NC * N_PAD, D), jnp.float32)),
        scratch_types=[
            pltpu.VMEM((N_PAD,), jnp.float32),
            pltpu.VMEM((W2, D), jnp.float32),
            pltpu.VMEM((W2, D), jnp.float32),
            pltpu.VMEM((W2,), jnp.int32),
            pltpu.VMEM((W2,), jnp.int32),
            pltpu.VMEM((W2,), jnp.int32),
            pltpu.VMEM((W2,), jnp.int32),
            pltpu.VMEM((W2,), jnp.float32),
            pltpu.VMEM((W2,), jnp.float32),
            pltpu.VMEM((W2,), jnp.float32),
            pltpu.VMEM((W2,), jnp.float32),
            pltpu.VMEM((W2,), jnp.int32),
            pltpu.VMEM((W2,), jnp.int32),
            pltpu.VMEM_SHARED((N_PAD, D), jnp.float32),
            pltpu.SemaphoreType.DMA,
            pltpu.SemaphoreType.DMA,
            pltpu.SemaphoreType.DMA,
            pltpu.SemaphoreType.DMA,
            pltpu.SemaphoreType.DMA,
            pltpu.SemaphoreType.DMA,
            pltpu.SemaphoreType.DMA,
            pltpu.SemaphoreType.DMA,
            pltpu.SemaphoreType.DMA,
        ])()
    return f(h, ex, dst_p, ir, dp.reshape(NC * N_PAD // D, D))


@jax.jit
def kernel(x, edge_index, edge_type, weight, q, k, bias):
    src = edge_index[0]
    dst = edge_index[1]
    pad = E_PAD - E
    ar = jnp.arange(pad, dtype=jnp.int32)
    src_p = jnp.concatenate([src, ar % N])
    dst_p = jnp.concatenate([dst, N + (ar % (N_PAD - N))])
    et_p = jnp.concatenate([edge_type, jnp.zeros((pad,), jnp.int32)])

    qh, kh = _tc_a1(x, weight, q, k)
    qhf = qh.reshape(N * R)
    khf = kh.reshape(N * R)

    ex, dp, ir = _sc1(qhf, khf, src_p, dst_p, et_p)
    h3 = _tc_a2(x, weight)
    h = h3.reshape(R * N, D)
    alpha_p, ap = _sc2(h, ex, dst_p, ir, dp)
    out = _tc_c(ap[:N], ap[N_PAD:N_PAD + N], x, bias.reshape(1, D))
    return out, alpha_p[:E].reshape(E, 1)


# speedup vs baseline: 1.0107x; 1.0107x over previous
import functools
import jax
import jax.numpy as jnp
from jax import lax
from jax.experimental import pallas as pl
from jax.experimental.pallas import tpu as pltpu
from jax.experimental.pallas import tpu_sc as plsc

N = 10000
D = 128
R = 19
E = 320000
NC = 2
NS = 16
LN = 16
NW = NC * NS
N_PAD = 10240
E_PAD = NW * 10240
EPT = E_PAD // NW
SUB = 128
W1 = 2048
W2 = 128
NBLK = 10
BN = N // NBLK


def _tca1_body(x_ref, w_ref, q_ref, k_ref, qh_ref, kh_ref):
    xb = x_ref[...]
    wf = w_ref[...].reshape(R * D, D)
    wq = jnp.dot(wf, q_ref[...], preferred_element_type=jnp.float32)
    wk = jnp.dot(wf, k_ref[...], preferred_element_type=jnp.float32)
    qh_ref[...] = jnp.dot(xb, wq.reshape(R, D).T,
                          preferred_element_type=jnp.float32)
    kh_ref[...] = jnp.dot(xb, wk.reshape(R, D).T,
                          preferred_element_type=jnp.float32)


def _tc_a1(x, weight, q, k):
    return pl.pallas_call(
        _tca1_body,
        grid=(NBLK,),
        in_specs=[
            pl.BlockSpec((BN, D), lambda i: (i, 0)),
            pl.BlockSpec((R, D, D), lambda i: (0, 0, 0)),
            pl.BlockSpec((D, 1), lambda i: (0, 0)),
            pl.BlockSpec((D, 1), lambda i: (0, 0)),
        ],
        out_specs=[
            pl.BlockSpec((BN, R), lambda i: (i, 0)),
            pl.BlockSpec((BN, R), lambda i: (i, 0)),
        ],
        out_shape=[
            jax.ShapeDtypeStruct((N, R), jnp.float32),
            jax.ShapeDtypeStruct((N, R), jnp.float32),
        ],
    )(x, weight, q, k)


def _tca2_body(x_ref, w_ref, h_ref):
    xb = x_ref[...]
    for r in range(R):
        h_ref[r] = jnp.dot(xb, w_ref[r], preferred_element_type=jnp.float32)


def _tc_a2(x, weight):
    return pl.pallas_call(
        _tca2_body,
        grid=(NBLK,),
        in_specs=[
            pl.BlockSpec((BN, D), lambda i: (i, 0)),
            pl.BlockSpec((R, D, D), lambda i: (0, 0, 0)),
        ],
        out_specs=pl.BlockSpec((R, BN, D), lambda i: (0, i, 0)),
        out_shape=jax.ShapeDtypeStruct((R, N, D), jnp.float32),
    )(x, weight)


def _tcc_body(a0_ref, a1_ref, x_ref, b_ref, o_ref):
    o_ref[...] = a0_ref[...] + a1_ref[...] + x_ref[...] + b_ref[...]


def _tc_c(a0, a1, x, brow):
    return pl.pallas_call(
        _tcc_body,
        grid=(NBLK,),
        in_specs=[
            pl.BlockSpec((BN, D), lambda i: (i, 0)),
            pl.BlockSpec((BN, D), lambda i: (i, 0)),
            pl.BlockSpec((BN, D), lambda i: (i, 0)),
            pl.BlockSpec((1, D), lambda i: (0, 0)),
        ],
        out_specs=pl.BlockSpec((BN, D), lambda i: (i, 0)),
        out_shape=jax.ShapeDtypeStruct((N, D), jnp.float32),
    )(a0, a1, x, brow)


def _sc1_body(qh_hbm, kh_hbm, src_hbm, dst_hbm, et_hbm,
              ex_hbm, dp_hbm, ir_hbm,
              src_v, dst_v, et_v, idxq_v, idxk_v, idxr_v, qv_v, kv_v, ex_v,
              dst2_v, ex2_v, zb_v, den_sh, sem, sem2):
    cid = lax.axis_index("c")
    sid = lax.axis_index("s")
    wid = cid * NS + sid
    base = wid * EPT
    zseg = N_PAD // NS

    for i in range(zseg // LN):
        zb_v[pl.ds(i * LN, LN)] = jnp.zeros((LN,), jnp.float32)
    pltpu.sync_copy(zb_v, den_sh.at[pl.ds(sid * zseg, zseg)])
    plsc.subcore_barrier()

    def window(g, carry):
        b0 = base + g * W1
        lcps = [pltpu.async_copy(src_hbm.at[pl.ds(b0, W1)], src_v, sem),
                pltpu.async_copy(dst_hbm.at[pl.ds(b0, W1)], dst_v, sem),
                pltpu.async_copy(et_hbm.at[pl.ds(b0, W1)], et_v, sem)]
        for c in lcps:
            c.wait()

        def idx_body(c, cr):
            s16 = src_v[pl.ds(c * LN, LN)]
            d16 = dst_v[pl.ds(c * LN, LN)]
            t16 = et_v[pl.ds(c * LN, LN)]
            dcl = jnp.where(d16 < N, d16, d16 - N)
            idxq_v[pl.ds(c * LN, LN)] = dcl * R + t16
            idxk_v[pl.ds(c * LN, LN)] = s16 * R + t16
            idxr_v[pl.ds(c * LN, LN)] = t16 * N + s16
            return cr
        lax.fori_loop(0, W1 // LN, idx_body, 0, unroll=4)

        cps = []
        for j in range(W1 // SUB):
            cps.append(pltpu.async_copy(
                qh_hbm.at[idxq_v.at[pl.ds(j * SUB, SUB)]],
                qv_v.at[pl.ds(j * SUB, SUB)], sem))
            cps.append(pltpu.async_copy(
                kh_hbm.at[idxk_v.at[pl.ds(j * SUB, SUB)]],
                kv_v.at[pl.ds(j * SUB, SUB)], sem))
        cps.append(pltpu.async_copy(idxr_v, ir_hbm.at[pl.ds(b0, W1)], sem))
        for c in cps:
            c.wait()

        def ex_body(c, cr):
            a = qv_v[pl.ds(c * LN, LN)] + kv_v[pl.ds(c * LN, LN)]
            a = jnp.where(a >= 0.0, a, a * jnp.float32(0.2))
            ex_v[pl.ds(c * LN, LN)] = jnp.exp(a)
            return cr
        lax.fori_loop(0, W1 // LN, ex_body, 0, unroll=4)

        pltpu.sync_copy(ex_v, ex_hbm.at[pl.ds(b0, W1)])

        @pl.when(g > 0)
        def _():
            for j in range(W1 // SUB):
                pltpu.make_async_copy(
                    ex2_v.at[j], den_sh.at[dst2_v.at[j]], sem2).wait()

        for j in range(W1 // SUB):
            for c in range(SUB // LN):
                off = j * SUB + c * LN
                dst2_v[j, pl.ds(c * LN, LN)] = dst_v[pl.ds(off, LN)]
                ex2_v[j, pl.ds(c * LN, LN)] = ex_v[pl.ds(off, LN)]
        for j in range(W1 // SUB):
            pltpu.async_copy(ex2_v.at[j], den_sh.at[dst2_v.at[j]], sem2,
                             add=True)
        return carry

    lax.fori_loop(0, EPT // W1, window, 0)
    for j in range(W1 // SUB):
        pltpu.make_async_copy(ex2_v.at[j], den_sh.at[dst2_v.at[j]],
                              sem2).wait()
    plsc.subcore_barrier()
    pltpu.sync_copy(den_sh.at[pl.ds(sid * zseg, zseg)],
                    dp_hbm.at[pl.ds(cid * N_PAD + sid * zseg, zseg)])


def _sc1(qhf, khf, src_p, dst_p, et_p):
    mesh = plsc.VectorSubcoreMesh(core_axis_name="c", subcore_axis_name="s")
    f = functools.partial(
        pl.kernel, _sc1_body, mesh=mesh,
        out_type=(jax.ShapeDtypeStruct((E_PAD,), jnp.float32),
                  jax.ShapeDtypeStruct((NC * N_PAD,), jnp.float32),
                  jax.ShapeDtypeStruct((E_PAD,), jnp.int32)),
        scratch_types=[
            pltpu.VMEM((W1,), jnp.int32),
            pltpu.VMEM((W1,), jnp.int32),
            pltpu.VMEM((W1,), jnp.int32),
            pltpu.VMEM((W1,), jnp.int32),
            pltpu.VMEM((W1,), jnp.int32),
            pltpu.VMEM((W1,), jnp.int32),
            pltpu.VMEM((W1,), jnp.float32),
            pltpu.VMEM((W1,), jnp.float32),
            pltpu.VMEM((W1,), jnp.float32),
            pltpu.VMEM((W1 // SUB, SUB), jnp.int32),
            pltpu.VMEM((W1 // SUB, SUB), jnp.float32),
            pltpu.VMEM((N_PAD // NS,), jnp.float32),
            pltpu.VMEM_SHARED((N_PAD,), jnp.float32),
            pltpu.SemaphoreType.DMA,
            pltpu.SemaphoreType.DMA,
        ])()
    return f(qhf, khf, src_p, dst_p, et_p)


def _sc2_body(h_hbm, ex_hbm, dst_hbm, ir_hbm, dp2_hbm,
              al_hbm, ap_hbm,
              dn_v, rows_a, rows_b, ir_va, ir_vb, dst_va, dst_vb,
              ex_va, ex_vb, al_va, al_vb, ds_va, ds_vb, agg_sh,
              semd, semg0, semg1, sems0, sems1, sema0, sema1, seml0, seml1):
    cid = lax.axis_index("c")
    sid = lax.axis_index("s")
    wid = cid * NS + sid
    base = wid * EPT
    zseg = N_PAD // NS
    nwin = EPT // W2
    npair = nwin // 2

    nr = N_PAD // D
    cdn = pltpu.async_copy(dp2_hbm.at[pl.ds(0, nr)],
                           rows_b.at[pl.ds(0, nr)], semd)
    pltpu.sync_copy(dp2_hbm.at[pl.ds(nr, nr)], rows_a.at[pl.ds(0, nr)])
    cdn.wait()

    def dn_body(r, cr):
        for cc in range(D // LN):
            o = pl.ds(cc * LN, LN)
            dn_v[pl.ds(r * D + cc * LN, LN)] = jnp.float32(1.0) / (
                rows_b[r, o] + rows_a[r, o] + jnp.float32(1e-16))
        return cr
    lax.fori_loop(0, nr, dn_body, 0)

    def z_body(i, cr):
        for c in range(D // LN):
            rows_a[i, pl.ds(c * LN, LN)] = jnp.zeros((LN,), jnp.float32)
        return cr
    lax.fori_loop(0, W2, z_body, 0)
    for off in range(0, zseg, W2):
        pltpu.sync_copy(rows_a, agg_sh.at[pl.ds(sid * zseg + off, W2)])
    plsc.subcore_barrier()

    rows = (rows_a, rows_b)
    ir_v = (ir_va, ir_vb)
    dst_v = (dst_va, dst_vb)
    ex_v = (ex_va, ex_vb)
    al_v = (al_va, al_vb)
    ds_v = (ds_va, ds_vb)
    semg = (semg0, semg1)
    sems = (sems0, sems1)
    sema = (sema0, sema1)
    seml = (seml0, seml1)

    def lin_fire(g, p):
        b0 = base + g * W2
        pltpu.async_copy(ir_hbm.at[pl.ds(b0, W2)], ir_v[p], seml[p])
        pltpu.async_copy(dst_hbm.at[pl.ds(b0, W2)], dst_v[p], seml[p])
        pltpu.async_copy(ex_hbm.at[pl.ds(b0, W2)], ex_v[p], seml[p])

    def lin_wait(p):
        pltpu.make_async_copy(ir_hbm.at[pl.ds(base, W2)], ir_v[p], seml[p]).wait()
        pltpu.make_async_copy(dst_hbm.at[pl.ds(base, W2)], dst_v[p], seml[p]).wait()
        pltpu.make_async_copy(ex_hbm.at[pl.ds(base, W2)], ex_v[p], seml[p]).wait()

    def gather_fire(p):
        pltpu.async_copy(h_hbm.at[ir_v[p]], rows[p], semg[p])

    def gather_wait(p):
        pltpu.make_async_copy(h_hbm.at[ir_v[p]], rows[p], semg[p]).wait()

    def scat_fire(p):
        pltpu.async_copy(rows[p], agg_sh.at[ds_v[p]], sems[p], add=True)

    def scat_wait(p):
        pltpu.make_async_copy(rows[p], agg_sh.at[ds_v[p]], sems[p]).wait()

    def al_fire(g, p):
        pltpu.async_copy(al_v[p], al_hbm.at[pl.ds(base + g * W2, W2)], sema[p])

    def al_wait(p):
        pltpu.make_async_copy(al_v[p], al_hbm.at[pl.ds(base, W2)],
                              sema[p]).wait()

    lin_fire(0, 0)
    lin_fire(1, 1)
    lin_wait(0)
    gather_fire(0)

    def pair(i, carry):
        for b in range(2):
            p = b
            g = 2 * i + b

            @pl.when(i > 0)
            def _():
                al_wait(p)

            for c in range(W2 // LN):
                d16 = dst_v[p][pl.ds(c * LN, LN)]
                inv = plsc.load_gather(dn_v, [d16])
                al_v[p][pl.ds(c * LN, LN)] = ex_v[p][pl.ds(c * LN, LN)] * inv
                ds_v[p][pl.ds(c * LN, LN)] = d16
            al_fire(g, p)

            gather_wait(p)
            if b == 0:
                @pl.when(i > 0)
                def _():
                    scat_wait(1)
                lin_wait(1)
                gather_fire(1)
            else:
                @pl.when(i < npair - 1)
                def _():
                    scat_wait(0)
                    lin_wait(0)
                    gather_fire(0)

            @pl.when(i < npair - 1)
            def _():
                lin_fire(g + 2, p)

            def s_body(r0, cr):
                s = plsc.load_gather(al_v[p], [jnp.full((LN,), 0, jnp.int32) + r0])
                for c in range(D // LN):
                    rows[p][r0, pl.ds(c * LN, LN)] = \
                        rows[p][r0, pl.ds(c * LN, LN)] * s
                return cr
            lax.fori_loop(0, W2, s_body, 0, unroll=4)
            scat_fire(p)
        return carry

    lax.fori_loop(0, npair, pair, 0)
    scat_wait(0)
    scat_wait(1)
    al_wait(0)
    al_wait(1)
    plsc.subcore_barrier()
    pltpu.sync_copy(agg_sh.at[pl.ds(sid * zseg, zseg)],
                    ap_hbm.at[pl.ds(cid * N_PAD + sid * zseg, zseg)])


def _sc2(h, ex, dst_p, ir, dp):
    mesh = plsc.VectorSubcoreMesh(core_axis_name="c", subcore_axis_name="s")
    f = functools.partial(
        pl.kernel, _sc2_body, mesh=mesh,
        compiler_params=pltpu.CompilerParams(needs_layout_passes=False),
        out_type=(jax.ShapeDtypeStruct((E_PAD,), jnp.float32),
                  jax.ShapeDtypeStruct((NC * N_PAD, D), jnp.float32)),
        scratch_types=[
            pltpu.VMEM((N_PAD,), jnp.float32),
            pltpu.VMEM((W2, D), jnp.float32),
            pltpu.VMEM((W2, D), jnp.float32),
            pltpu.VMEM((W2,), jnp.int32),
            pltpu.VMEM((W2,), jnp.int32),
            pltpu.VMEM((W2,), jnp.int32),
            pltpu.VMEM((W2,), jnp.int32),
            pltpu.VMEM((W2,), jnp.float32),
            pltpu.VMEM((W2,), jnp.float32),
            pltpu.VMEM((W2,), jnp.float32),
            pltpu.VMEM((W2,), jnp.float32),
            pltpu.VMEM((W2,), jnp.int32),
            pltpu.VMEM((W2,), jnp.int32),
            pltpu.VMEM_SHARED((N_PAD, D), jnp.float32),
            pltpu.SemaphoreType.DMA,
            pltpu.SemaphoreType.DMA,
            pltpu.SemaphoreType.DMA,
            pltpu.SemaphoreType.DMA,
            pltpu.SemaphoreType.DMA,
            pltpu.SemaphoreType.DMA,
            pltpu.SemaphoreType.DMA,
            pltpu.SemaphoreType.DMA,
            pltpu.SemaphoreType.DMA,
        ])()
    return f(h, ex, dst_p, ir, dp.reshape(NC * N_PAD // D, D))


@jax.jit
def kernel(x, edge_index, edge_type, weight, q, k, bias):
    src = edge_index[0]
    dst = edge_index[1]
    pad = E_PAD - E
    ar = jnp.arange(pad, dtype=jnp.int32)
    src_p = jnp.concatenate([src, ar % N])
    dst_p = jnp.concatenate([dst, N + (ar % (N_PAD - N))])
    et_p = jnp.concatenate([edge_type, jnp.zeros((pad,), jnp.int32)])

    qh, kh = _tc_a1(x, weight, q, k)
    qhf = qh.reshape(N * R)
    khf = kh.reshape(N * R)

    ex, dp, ir = _sc1(qhf, khf, src_p, dst_p, et_p)
    h3 = _tc_a2(x, weight)
    h = h3.reshape(R * N, D)
    alpha_p, ap = _sc2(h, ex, dst_p, ir, dp)
    out = _tc_c(ap[:N], ap[N_PAD:N_PAD + N], x, bias.reshape(1, D))
    return out, alpha_p[:E].reshape(E, 1)
